# baseline (speedup 1.0000x reference)
import functools

import jax
import jax.numpy as jnp
from jax.experimental import pallas as pl
from jax.experimental.pallas import tpu as pltpu
from jax.experimental.pallas import tpu_sc as plsc


def _embed_body(x_ref, w_ref, out_ref):
    out_ref[...] = jnp.dot(x_ref[...], w_ref[...],
                           preferred_element_type=jnp.float32)


def _embed(x, w, block=512):
    n, d_in = x.shape
    d_out = w.shape[1]
    return pl.pallas_call(
        _embed_body,
        grid=(n // block,),
        in_specs=[
            pl.BlockSpec((block, d_in), lambda i: (i, 0)),
            pl.BlockSpec((d_in, d_out), lambda i: (0, 0)),
        ],
        out_specs=pl.BlockSpec((block, d_out), lambda i: (i, 0)),
        out_shape=jax.ShapeDtypeStruct((n, d_out), jnp.float32),
    )(x, w)


def _expand_rows_sc(vec, n, w, row0, nrows):
    info = plsc.get_sparse_core_info()
    nc, ns = info.num_cores, info.num_subcores
    nw = nc * ns
    rows_per = nrows // nw
    nbuf = 8
    mesh = plsc.VectorSubcoreMesh(core_axis_name="c", subcore_axis_name="s")

    @functools.partial(
        pl.kernel,
        out_type=jax.ShapeDtypeStruct((nrows * w,), jnp.float32),
        mesh=mesh,
        scratch_types=([pltpu.VMEM((w,), jnp.float32)] * nbuf
                       + [pltpu.SemaphoreType.DMA] * nbuf
                       + [pltpu.SemaphoreType.DMA] * nbuf),
    )
    def k(vec_hbm, out_hbm, *scratch):
        bufs = scratch[:nbuf]
        sa = scratch[nbuf:2 * nbuf]
        sb = scratch[2 * nbuf:]
        wid = jax.lax.axis_index("s") * nc + jax.lax.axis_index("c")
        base = wid * rows_per

        def _in(r, b, do_wait):
            i = row0 + base + r
            off = i * (i + 1) // 2
            a = (off // 8) * 8

            @pl.when(i != n - 1)
            def _():
                c = pltpu.make_async_copy(vec_hbm.at[pl.ds(a, w)],
                                          bufs[b], sa[b])
                c.wait() if do_wait else c.start()

            @pl.when(i == n - 1)
            def _():
                c = pltpu.make_async_copy(vec_hbm.at[pl.ds(a, w - 8)],
                                          bufs[b].at[pl.ds(0, w - 8)], sa[b])
                c.wait() if do_wait else c.start()

        def mk_out(r, b):
            return pltpu.make_async_copy(
                bufs[b], out_hbm.at[pl.ds((base + r) * w, w)], sb[b])

        for b in range(nbuf):
            _in(b, b, False)

        def group(g, _):
            for b in range(nbuf):
                r = g * nbuf + b
                _in(r, b, True)
                mk_out(r, b).start()
                nxt = r + nbuf

                @pl.when(nxt < rows_per)
                def _(b=b, r=r, nxt=nxt):
                    mk_out(r, b).wait()
                    _in(nxt, b, False)

            return 0

        jax.lax.fori_loop(0, rows_per // nbuf, group, 0)

        for b in range(nbuf):
            mk_out(rows_per - nbuf + b, b).wait()

    return k(vec).reshape(nrows, w)


def _atilde_part(graph, mask, lhalf, bi0, bj0, nbi, nbj, block):
    n = graph.shape[0]
    t = block
    mx0 = max(bi0, bj0)

    def body(g_ref, m_ref, lmain_ref, lext_ref, at_ref, d_ref):
        bi = bi0 + pl.program_id(0)
        bj = bj0 + pl.program_id(1)
        mx = jnp.maximum(bi, bj)

        rows_l = mx * t + jax.lax.broadcasted_iota(jnp.int32, (t, 1), 0)
        svec = (rows_l * (rows_l + 1) // 2) % 8
        allp = jnp.concatenate([lmain_ref[...], lext_ref[:, :8]], axis=1)
        x4 = jnp.where((svec & 4) != 0, allp[:, 4:t + 8], allp[:, 0:t + 4])
        x2 = jnp.where((svec & 2) != 0, x4[:, 2:t + 4], x4[:, 0:t + 2])
        shifted = jnp.where((svec & 1) != 0, x2[:, 1:t + 1], x2[:, 0:t])

        ssig = jax.nn.sigmoid(shifted)
        g = g_ref[...]
        m = m_ref[...]

        @pl.when(bi > bj)
        def _():
            at_ref[...] = (g * jnp.where(m, ssig, jnp.float32(1.0))
                           ).astype(jnp.bfloat16)

        @pl.when(bi < bj)
        def _():
            at_ref[...] = (g * jnp.where(m, ssig.T, jnp.float32(1.0))
                           ).astype(jnp.bfloat16)

        @pl.when(bi == bj)
        def _():
            lo = (jax.lax.broadcasted_iota(jnp.int32, (t, t), 1)
                  <= jax.lax.broadcasted_iota(jnp.int32, (t, t), 0))
            psig = jnp.where(lo, ssig, ssig.T)
            at_ref[...] = (g * jnp.where(m, psig, jnp.float32(1.0))
                           ).astype(jnp.bfloat16)

        partial = jnp.sum(at_ref[...].astype(jnp.float32), axis=1,
                          keepdims=True)

        @pl.when(pl.program_id(1) == 0)
        def _():
            d_ref[...] = partial

        @pl.when(pl.program_id(1) != 0)
        def _():
            d_ref[...] += partial

    def _lmain(i, j):
        bi, bj = bi0 + i, bj0 + j
        lower = bi >= bj
        return (jnp.where(lower, bi, bj) - mx0, jnp.where(lower, bj, bi))

    def _lext(i, j):
        bi, bj = bi0 + i, bj0 + j
        lower = bi >= bj
        mn = jnp.where(lower, bj, bi)
        return (jnp.where(lower, bi, bj) - mx0, (mn + 1) * (block // 128))

    return pl.pallas_call(
        body,
        grid=(nbi, nbj),
        in_specs=[
            pl.BlockSpec((block, block), lambda i, j: (bi0 + i, bj0 + j)),
            pl.BlockSpec((block, block), lambda i, j: (bi0 + i, bj0 + j)),
            pl.BlockSpec((block, block), _lmain),
            pl.BlockSpec((block, 128), _lext),
        ],
        out_specs=[
            pl.BlockSpec((block, block), lambda i, j: (i, j)),
            pl.BlockSpec((block, 1), lambda i, j: (i, 0)),
        ],
        out_shape=[
            jax.ShapeDtypeStruct((nbi * block, nbj * block), jnp.bfloat16),
            jax.ShapeDtypeStruct((nbi * block, 1), jnp.float32),
        ],
    )(graph, mask, lhalf, lhalf)


def _d_full(dga_ref, dgr_ref, dgb_ref):
    top = dga_ref[...] + dgr_ref[...]
    deg = jnp.concatenate([top, dgb_ref[...]], axis=0)
    return jax.lax.rsqrt(deg + jnp.float32(1e-7))


def _spmm_top_body(ata_ref, atr_ref, emb_ref, dga_ref, dgr_ref, dgb_ref,
                   dra_ref, drr_ref, wd_ref, out_ref, y_ref):
    h = ata_ref.shape[1]

    @pl.when(pl.program_id(0) == 0)
    def _():
        y_ref[...] = emb_ref[...] * _d_full(dga_ref, dgr_ref, dgb_ref)

    acc = (jnp.dot(ata_ref[...], y_ref[0:h], preferred_element_type=jnp.float32)
           + jnp.dot(atr_ref[...], y_ref[h:2 * h],
                     preferred_element_type=jnp.float32))
    d_r = jax.lax.rsqrt(dra_ref[...] + drr_ref[...] + jnp.float32(1e-7))
    hid = jnp.maximum(acc * d_r, jnp.float32(0.0))
    out_ref[...] = jnp.dot(hid, wd_ref[...].T,
                           preferred_element_type=jnp.float32)


def _spmm_bot_body(atb_ref, emb_ref, dga_ref, dgr_ref, dgb_ref,
                   drb_ref, wd_ref, out_ref, y_ref):
    @pl.when(pl.program_id(0) == 0)
    def _():
        y_ref[...] = emb_ref[...] * _d_full(dga_ref, dgr_ref, dgb_ref)

    acc = jnp.dot(atb_ref[...], y_ref[...], preferred_element_type=jnp.float32)
    d_r = jax.lax.rsqrt(drb_ref[...] + jnp.float32(1e-7))
    hid = jnp.maximum(acc * d_r, jnp.float32(0.0))
    out_ref[...] = jnp.dot(hid, wd_ref[...].T,
                           preferred_element_type=jnp.float32)


def _spmm_head(at_a, at_r, at_b, emb, deg_a, deg_r, deg_b, w_dense,
               block=256):
    n, d_gcn = emb.shape
    h = n // 2
    d_out = w_dense.shape[0]
    common = [
        pl.BlockSpec((n, d_gcn), lambda i: (0, 0)),
        pl.BlockSpec((h, 1), lambda i: (0, 0)),
        pl.BlockSpec((h, 1), lambda i: (0, 0)),
        pl.BlockSpec((h, 1), lambda i: (0, 0)),
    ]
    top = pl.pallas_call(
        _spmm_top_body,
        grid=(h // block,),
        in_specs=[
            pl.BlockSpec((block, h), lambda i: (i, 0)),
            pl.BlockSpec((block, h), lambda i: (i, 0)),
            *common,
            pl.BlockSpec((block, 1), lambda i: (i, 0)),
            pl.BlockSpec((block, 1), lambda i: (i, 0)),
            pl.BlockSpec((d_out, d_gcn), lambda i: (0, 0)),
        ],
        out_specs=pl.BlockSpec((block, d_out), lambda i: (i, 0)),
        out_shape=jax.ShapeDtypeStruct((h, d_out), jnp.float32),
        scratch_shapes=[pltpu.VMEM((n, d_gcn), jnp.float32)],
    )(at_a, at_r, emb, deg_a, deg_r, deg_b, deg_a, deg_r, w_dense)
    bot = pl.pallas_call(
        _spmm_bot_body,
        grid=(h // block,),
        in_specs=[
            pl.BlockSpec((block, n), lambda i: (i, 0)),
            *common,
            pl.BlockSpec((block, 1), lambda i: (i, 0)),
            pl.BlockSpec((d_out, d_gcn), lambda i: (0, 0)),
        ],
        out_specs=pl.BlockSpec((block, d_out), lambda i: (i, 0)),
        out_shape=jax.ShapeDtypeStruct((h, d_out), jnp.float32),
        scratch_shapes=[pltpu.VMEM((n, d_gcn), jnp.float32)],
    )(at_b, emb, deg_a, deg_r, deg_b, deg_b, w_dense)
    return top, bot


def kernel(user_X, item_X, graph_A, mask_sub_adj, P_symm, W_gcn, W_dense):
    num_user = user_X.shape[0]
    x = jnp.concatenate([user_X, item_X], axis=0)
    n = x.shape[0]
    w = n + 8
    block = next(b for b in (1024, 768, 512, 256, 128) if (n // 2) % b == 0)
    nb = n // block
    hb = nb // 2
    h = n // 2

    emb = _embed(x, W_gcn)
    lpad_a = _expand_rows_sc(P_symm, n, w, 0, h)
    lpad_b = _expand_rows_sc(P_symm, n, w, h, h)

    at_a, deg_a = _atilde_part(graph_A, mask_sub_adj, lpad_a,
                               0, 0, hb, hb, block)
    at_b, deg_b = _atilde_part(graph_A, mask_sub_adj, lpad_b,
                               hb, 0, hb, nb, block)
    at_r, deg_r = _atilde_part(graph_A, mask_sub_adj, lpad_b,
                               0, hb, hb, hb, block)

    top, bot = _spmm_head(at_a, at_r, at_b, emb, deg_a, deg_r, deg_b,
                          W_dense)
    out = jnp.concatenate([top, bot], axis=0)
    return out[:num_user], out[num_user:]


# speedup vs baseline: 1.0411x; 1.0411x over previous
import functools

import jax
import jax.numpy as jnp
from jax.experimental import pallas as pl
from jax.experimental.pallas import tpu as pltpu
from jax.experimental.pallas import tpu_sc as plsc


def _embed_body(x_ref, w_ref, out_ref):
    out_ref[...] = jnp.dot(x_ref[...], w_ref[...],
                           preferred_element_type=jnp.float32)


def _embed(x, w, block=512):
    n, d_in = x.shape
    d_out = w.shape[1]
    return pl.pallas_call(
        _embed_body,
        grid=(n // block,),
        in_specs=[
            pl.BlockSpec((block, d_in), lambda i: (i, 0)),
            pl.BlockSpec((d_in, d_out), lambda i: (0, 0)),
        ],
        out_specs=pl.BlockSpec((block, d_out), lambda i: (i, 0)),
        out_shape=jax.ShapeDtypeStruct((n, d_out), jnp.float32),
    )(x, w)


def _expand_rows_sc(vec, n, w, row0, nrows, dep=None):
    info = plsc.get_sparse_core_info()
    nc, ns = info.num_cores, info.num_subcores
    nw = nc * ns
    rows_per = nrows // nw
    nbuf = 8
    mesh = plsc.VectorSubcoreMesh(core_axis_name="c", subcore_axis_name="s")

    @functools.partial(
        pl.kernel,
        out_type=jax.ShapeDtypeStruct((nrows * w,), jnp.float32),
        mesh=mesh,
        scratch_types=([pltpu.VMEM((w,), jnp.float32)] * nbuf
                       + [pltpu.SemaphoreType.DMA] * nbuf
                       + [pltpu.SemaphoreType.DMA] * nbuf),
    )
    def k(vec_hbm, dep_hbm, out_hbm, *scratch):
        del dep_hbm
        bufs = scratch[:nbuf]
        sa = scratch[nbuf:2 * nbuf]
        sb = scratch[2 * nbuf:]
        wid = jax.lax.axis_index("s") * nc + jax.lax.axis_index("c")
        base = wid * rows_per

        def _in(r, b, do_wait):
            i = row0 + base + r
            off = i * (i + 1) // 2
            a = (off // 8) * 8

            @pl.when(i != n - 1)
            def _():
                c = pltpu.make_async_copy(vec_hbm.at[pl.ds(a, w)],
                                          bufs[b], sa[b])
                c.wait() if do_wait else c.start()

            @pl.when(i == n - 1)
            def _():
                c = pltpu.make_async_copy(vec_hbm.at[pl.ds(a, w - 8)],
                                          bufs[b].at[pl.ds(0, w - 8)], sa[b])
                c.wait() if do_wait else c.start()

        def mk_out(r, b):
            return pltpu.make_async_copy(
                bufs[b], out_hbm.at[pl.ds((base + r) * w, w)], sb[b])

        for b in range(nbuf):
            _in(b, b, False)

        def group(g, _):
            for b in range(nbuf):
                r = g * nbuf + b
                _in(r, b, True)
                mk_out(r, b).start()
                nxt = r + nbuf

                @pl.when(nxt < rows_per)
                def _(b=b, r=r, nxt=nxt):
                    mk_out(r, b).wait()
                    _in(nxt, b, False)

            return 0

        jax.lax.fori_loop(0, rows_per // nbuf, group, 0)

        for b in range(nbuf):
            mk_out(rows_per - nbuf + b, b).wait()

    if dep is None:
        dep = jnp.zeros((8,), jnp.float32)
    return k(vec, dep)


def _atilde_part(graph, mask, lhalf, bi0, bj0, nbi, nbj, block):
    n = graph.shape[0]
    t = block
    mx0 = max(bi0, bj0)

    def body(g_ref, m_ref, lmain_ref, lext_ref, at_ref, d_ref):
        bi = bi0 + pl.program_id(0)
        bj = bj0 + pl.program_id(1)
        mx = jnp.maximum(bi, bj)

        rows_l = mx * t + jax.lax.broadcasted_iota(jnp.int32, (t, 1), 0)
        svec = (rows_l * (rows_l + 1) // 2) % 8
        allp = jnp.concatenate([lmain_ref[...], lext_ref[:, :8]], axis=1)
        x4 = jnp.where((svec & 4) != 0, allp[:, 4:t + 8], allp[:, 0:t + 4])
        x2 = jnp.where((svec & 2) != 0, x4[:, 2:t + 4], x4[:, 0:t + 2])
        shifted = jnp.where((svec & 1) != 0, x2[:, 1:t + 1], x2[:, 0:t])

        ssig = jax.nn.sigmoid(shifted)
        g = g_ref[...]
        m = m_ref[...] != 0

        @pl.when(bi > bj)
        def _():
            at_ref[...] = (g * jnp.where(m, ssig, jnp.float32(1.0))
                           ).astype(jnp.bfloat16)

        @pl.when(bi < bj)
        def _():
            at_ref[...] = (g * jnp.where(m, ssig.T, jnp.float32(1.0))
                           ).astype(jnp.bfloat16)

        @pl.when(bi == bj)
        def _():
            lo = (jax.lax.broadcasted_iota(jnp.int32, (t, t), 1)
                  <= jax.lax.broadcasted_iota(jnp.int32, (t, t), 0))
            psig = jnp.where(lo, ssig, ssig.T)
            at_ref[...] = (g * jnp.where(m, psig, jnp.float32(1.0))
                           ).astype(jnp.bfloat16)

        partial = jnp.sum(at_ref[...].astype(jnp.float32), axis=1,
                          keepdims=True)

        @pl.when(pl.program_id(1) == 0)
        def _():
            d_ref[...] = partial

        @pl.when(pl.program_id(1) != 0)
        def _():
            d_ref[...] += partial

    def _lmain(i, j):
        bi, bj = bi0 + i, bj0 + j
        lower = bi >= bj
        return (jnp.where(lower, bi, bj) - mx0, jnp.where(lower, bj, bi))

    def _lext(i, j):
        bi, bj = bi0 + i, bj0 + j
        lower = bi >= bj
        mn = jnp.where(lower, bj, bi)
        return (jnp.where(lower, bi, bj) - mx0, (mn + 1) * (block // 128))

    return pl.pallas_call(
        body,
        grid=(nbi, nbj),
        in_specs=[
            pl.BlockSpec((block, block), lambda i, j: (bi0 + i, bj0 + j)),
            pl.BlockSpec((block, block), lambda i, j: (bi0 + i, bj0 + j)),
            pl.BlockSpec((block, block), _lmain),
            pl.BlockSpec((block, 128), _lext),
        ],
        out_specs=[
            pl.BlockSpec((block, block), lambda i, j: (i, j)),
            pl.BlockSpec((block, 1), lambda i, j: (i, 0)),
        ],
        out_shape=[
            jax.ShapeDtypeStruct((nbi * block, nbj * block), jnp.bfloat16),
            jax.ShapeDtypeStruct((nbi * block, 1), jnp.float32),
        ],
    )(graph, mask, lhalf, lhalf)


def _d_full(dga_ref, dgr_ref, dgb_ref):
    top = dga_ref[...] + dgr_ref[...]
    deg = jnp.concatenate([top, dgb_ref[...]], axis=0)
    return jax.lax.rsqrt(deg + jnp.float32(1e-7))


def _spmm_top_body(ata_ref, atr_ref, emb_ref, dga_ref, dgr_ref, dgb_ref,
                   dra_ref, drr_ref, wd_ref, out_ref, y_ref):
    h = ata_ref.shape[1]

    @pl.when(pl.program_id(0) == 0)
    def _():
        y_ref[...] = emb_ref[...] * _d_full(dga_ref, dgr_ref, dgb_ref)

    acc = (jnp.dot(ata_ref[...], y_ref[0:h], preferred_element_type=jnp.float32)
           + jnp.dot(atr_ref[...], y_ref[h:2 * h],
                     preferred_element_type=jnp.float32))
    d_r = jax.lax.rsqrt(dra_ref[...] + drr_ref[...] + jnp.float32(1e-7))
    hid = jnp.maximum(acc * d_r, jnp.float32(0.0))
    out_ref[...] = jnp.dot(hid, wd_ref[...].T,
                           preferred_element_type=jnp.float32)


def _spmm_bot_body(atb_ref, emb_ref, dga_ref, dgr_ref, dgb_ref,
                   drb_ref, wd_ref, out_ref, y_ref):
    @pl.when(pl.program_id(0) == 0)
    def _():
        y_ref[...] = emb_ref[...] * _d_full(dga_ref, dgr_ref, dgb_ref)

    acc = jnp.dot(atb_ref[...], y_ref[...], preferred_element_type=jnp.float32)
    d_r = jax.lax.rsqrt(drb_ref[...] + jnp.float32(1e-7))
    hid = jnp.maximum(acc * d_r, jnp.float32(0.0))
    out_ref[...] = jnp.dot(hid, wd_ref[...].T,
                           preferred_element_type=jnp.float32)


def _spmm_head(at_a, at_r, at_b, emb, deg_a, deg_r, deg_b, w_dense,
               block=256):
    n, d_gcn = emb.shape
    h = n // 2
    d_out = w_dense.shape[0]
    common = [
        pl.BlockSpec((n, d_gcn), lambda i: (0, 0)),
        pl.BlockSpec((h, 1), lambda i: (0, 0)),
        pl.BlockSpec((h, 1), lambda i: (0, 0)),
        pl.BlockSpec((h, 1), lambda i: (0, 0)),
    ]
    top = pl.pallas_call(
        _spmm_top_body,
        grid=(h // block,),
        in_specs=[
            pl.BlockSpec((block, h), lambda i: (i, 0)),
            pl.BlockSpec((block, h), lambda i: (i, 0)),
            *common,
            pl.BlockSpec((block, 1), lambda i: (i, 0)),
            pl.BlockSpec((block, 1), lambda i: (i, 0)),
            pl.BlockSpec((d_out, d_gcn), lambda i: (0, 0)),
        ],
        out_specs=pl.BlockSpec((block, d_out), lambda i: (i, 0)),
        out_shape=jax.ShapeDtypeStruct((h, d_out), jnp.float32),
        scratch_shapes=[pltpu.VMEM((n, d_gcn), jnp.float32)],
    )(at_a, at_r, emb, deg_a, deg_r, deg_b, deg_a, deg_r, w_dense)
    bot = pl.pallas_call(
        _spmm_bot_body,
        grid=(h // block,),
        in_specs=[
            pl.BlockSpec((block, n), lambda i: (i, 0)),
            *common,
            pl.BlockSpec((block, 1), lambda i: (i, 0)),
            pl.BlockSpec((d_out, d_gcn), lambda i: (0, 0)),
        ],
        out_specs=pl.BlockSpec((block, d_out), lambda i: (i, 0)),
        out_shape=jax.ShapeDtypeStruct((h, d_out), jnp.float32),
        scratch_shapes=[pltpu.VMEM((n, d_gcn), jnp.float32)],
    )(at_b, emb, deg_a, deg_r, deg_b, deg_b, w_dense)
    return top, bot


def kernel(user_X, item_X, graph_A, mask_sub_adj, P_symm, W_gcn, W_dense):
    num_user = user_X.shape[0]
    x = jnp.concatenate([user_X, item_X], axis=0)
    n = x.shape[0]
    w = n + 8
    block = next(b for b in (1024, 768, 512, 256, 128) if (n // 2) % b == 0)
    nb = n // block
    hb = nb // 2
    h = n // 2

    emb = _embed(x, W_gcn)
    mask8 = mask_sub_adj.astype(jnp.int8)
    lpad_a_flat = _expand_rows_sc(P_symm, n, w, 0, h)
    lpad_b_flat = _expand_rows_sc(P_symm, n, w, h, h,
                                  dep=jax.lax.slice(lpad_a_flat, (0,), (8,)))
    lpad_a = lpad_a_flat.reshape(h, w)
    lpad_b = lpad_b_flat.reshape(h, w)

    at_a, deg_a = _atilde_part(graph_A, mask8, lpad_a,
                               0, 0, hb, hb, block)
    at_b, deg_b = _atilde_part(graph_A, mask8, lpad_b,
                               hb, 0, hb, nb, block)
    at_r, deg_r = _atilde_part(graph_A, mask8, lpad_b,
                               0, hb, hb, hb, block)

    top, bot = _spmm_head(at_a, at_r, at_b, emb, deg_a, deg_r, deg_b,
                          W_dense)
    out = jnp.concatenate([top, bot], axis=0)
    return out[:num_user], out[num_user:]


# speedup vs baseline: 1.1226x; 1.0783x over previous
import functools

import jax
import jax.numpy as jnp
from jax.experimental import pallas as pl
from jax.experimental.pallas import tpu as pltpu
from jax.experimental.pallas import tpu_sc as plsc


def _embed_body(x_ref, w_ref, out_ref):
    out_ref[...] = jnp.dot(x_ref[...], w_ref[...],
                           preferred_element_type=jnp.float32)


def _embed(x, w, block=512):
    n, d_in = x.shape
    d_out = w.shape[1]
    return pl.pallas_call(
        _embed_body,
        grid=(n // block,),
        in_specs=[
            pl.BlockSpec((block, d_in), lambda i: (i, 0)),
            pl.BlockSpec((d_in, d_out), lambda i: (0, 0)),
        ],
        out_specs=pl.BlockSpec((block, d_out), lambda i: (i, 0)),
        out_shape=jax.ShapeDtypeStruct((n, d_out), jnp.float32),
    )(x, w)


def _expand_rows_sc(vec, n, w, row0, nrows, tile, dep=None):
    info = plsc.get_sparse_core_info()
    nc, ns = info.num_cores, info.num_subcores
    nw = nc * ns
    rows_per = nrows // nw
    nbuf = 8
    mesh = plsc.VectorSubcoreMesh(core_axis_name="c", subcore_axis_name="s")

    @functools.partial(
        pl.kernel,
        out_type=jax.ShapeDtypeStruct((nrows * w,), jnp.float32),
        mesh=mesh,
        scratch_types=([pltpu.VMEM((w,), jnp.float32)] * nbuf
                       + [pltpu.SemaphoreType.DMA] * nbuf
                       + [pltpu.SemaphoreType.DMA] * nbuf),
    )
    def k(vec_hbm, dep_hbm, out_hbm, *scratch):
        del dep_hbm
        bufs = scratch[:nbuf]
        sa = scratch[nbuf:2 * nbuf]
        sb = scratch[2 * nbuf:]
        wid = jax.lax.axis_index("s") * nc + jax.lax.axis_index("c")
        base = wid * rows_per

        buckets = range(row0 // tile, (row0 + nrows - 1) // tile + 1)

        def _buck(r, b, do_wait, mk):
            i = row0 + base + r
            for blk in buckets:
                wl = min((blk + 1) * tile + 8, w)
                short = wl == w

                @pl.when((i // tile == blk) & (i != n - 1))
                def _(wl=wl):
                    c = mk(r, b, i, wl)
                    c.wait() if do_wait else c.start()

                if short:
                    @pl.when((i // tile == blk) & (i == n - 1))
                    def _(wl=wl):
                        c = mk(r, b, i, wl - 8)
                        c.wait() if do_wait else c.start()

        def mk_in(r, b, i, wl):
            off = i * (i + 1) // 2
            a = (off // 8) * 8
            return pltpu.make_async_copy(vec_hbm.at[pl.ds(a, wl)],
                                         bufs[b].at[pl.ds(0, wl)], sa[b])

        def mk_out(r, b, i, wl):
            return pltpu.make_async_copy(
                bufs[b].at[pl.ds(0, wl)],
                out_hbm.at[pl.ds((base + r) * w, wl)], sb[b])

        for b in range(nbuf):
            _buck(b, b, False, mk_in)

        def group(g, _):
            for b in range(nbuf):
                r = g * nbuf + b
                _buck(r, b, True, mk_in)
                _buck(r, b, False, mk_out)
                nxt = r + nbuf

                @pl.when(nxt < rows_per)
                def _(b=b, r=r, nxt=nxt):
                    _buck(r, b, True, mk_out)
                    _buck(nxt, b, False, mk_in)

            return 0

        jax.lax.fori_loop(0, rows_per // nbuf, group, 0)

        for b in range(nbuf):
            _buck(rows_per - nbuf + b, b, True, mk_out)

    if dep is None:
        dep = jnp.zeros((8,), jnp.float32)
    return k(vec, dep)


def _atilde_part(graph, mask, lhalf, bi0, bj0, nbi, nbj, block):
    n = graph.shape[0]
    t = block
    mx0 = max(bi0, bj0)

    def body(g_ref, m_ref, lmain_ref, lext_ref, at_ref, d_ref):
        bi = bi0 + pl.program_id(0)
        bj = bj0 + pl.program_id(1)
        mx = jnp.maximum(bi, bj)

        rows_l = mx * t + jax.lax.broadcasted_iota(jnp.int32, (t, 1), 0)
        svec = (rows_l * (rows_l + 1) // 2) % 8
        allp = jnp.concatenate([lmain_ref[...], lext_ref[:, :8]], axis=1)
        x4 = jnp.where((svec & 4) != 0, allp[:, 4:t + 8], allp[:, 0:t + 4])
        x2 = jnp.where((svec & 2) != 0, x4[:, 2:t + 4], x4[:, 0:t + 2])
        shifted = jnp.where((svec & 1) != 0, x2[:, 1:t + 1], x2[:, 0:t])

        ssig = jax.nn.sigmoid(shifted)
        g = g_ref[...]
        m = m_ref[...] != 0

        def emit(psig):
            at = g * jnp.where(m, psig, jnp.float32(1.0))
            at_ref[...] = at.astype(jnp.bfloat16)
            p = jnp.sum(at, axis=1, keepdims=True)

            @pl.when(pl.program_id(1) == 0)
            def _():
                d_ref[...] = p

            @pl.when(pl.program_id(1) != 0)
            def _():
                d_ref[...] += p

        @pl.when(bi > bj)
        def _():
            emit(ssig)

        @pl.when(bi < bj)
        def _():
            emit(ssig.T)

        @pl.when(bi == bj)
        def _():
            lo = (jax.lax.broadcasted_iota(jnp.int32, (t, t), 1)
                  <= jax.lax.broadcasted_iota(jnp.int32, (t, t), 0))
            emit(jnp.where(lo, ssig, ssig.T))

    def _lmain(i, j):
        bi, bj = bi0 + i, bj0 + j
        lower = bi >= bj
        return (jnp.where(lower, bi, bj) - mx0, jnp.where(lower, bj, bi))

    def _lext(i, j):
        bi, bj = bi0 + i, bj0 + j
        lower = bi >= bj
        mn = jnp.where(lower, bj, bi)
        return (jnp.where(lower, bi, bj) - mx0, (mn + 1) * (block // 128))

    return pl.pallas_call(
        body,
        grid=(nbi, nbj),
        in_specs=[
            pl.BlockSpec((block, block), lambda i, j: (bi0 + i, bj0 + j)),
            pl.BlockSpec((block, block), lambda i, j: (bi0 + i, bj0 + j)),
            pl.BlockSpec((block, block), _lmain),
            pl.BlockSpec((block, 128), _lext),
        ],
        out_specs=[
            pl.BlockSpec((block, block), lambda i, j: (i, j)),
            pl.BlockSpec((block, 1), lambda i, j: (i, 0)),
        ],
        out_shape=[
            jax.ShapeDtypeStruct((nbi * block, nbj * block), jnp.bfloat16),
            jax.ShapeDtypeStruct((nbi * block, 1), jnp.float32),
        ],
    )(graph, mask, lhalf, lhalf)


def _d_full(dga_ref, dgr_ref, dgb_ref):
    top = dga_ref[...] + dgr_ref[...]
    deg = jnp.concatenate([top, dgb_ref[...]], axis=0)
    return jax.lax.rsqrt(deg + jnp.float32(1e-7))


def _spmm_top_body(ata_ref, atr_ref, emb_ref, dga_ref, dgr_ref, dgb_ref,
                   dra_ref, drr_ref, wd_ref, out_ref, y_ref):
    h = ata_ref.shape[1]

    @pl.when(pl.program_id(0) == 0)
    def _():
        y_ref[...] = emb_ref[...] * _d_full(dga_ref, dgr_ref, dgb_ref)

    acc = (jnp.dot(ata_ref[...], y_ref[0:h], preferred_element_type=jnp.float32)
           + jnp.dot(atr_ref[...], y_ref[h:2 * h],
                     preferred_element_type=jnp.float32))
    d_r = jax.lax.rsqrt(dra_ref[...] + drr_ref[...] + jnp.float32(1e-7))
    hid = jnp.maximum(acc * d_r, jnp.float32(0.0))
    out_ref[...] = jnp.dot(hid, wd_ref[...].T,
                           preferred_element_type=jnp.float32)


def _spmm_bot_body(atb_ref, emb_ref, dga_ref, dgr_ref, dgb_ref,
                   drb_ref, wd_ref, out_ref, y_ref):
    @pl.when(pl.program_id(0) == 0)
    def _():
        y_ref[...] = emb_ref[...] * _d_full(dga_ref, dgr_ref, dgb_ref)

    acc = jnp.dot(atb_ref[...], y_ref[...], preferred_element_type=jnp.float32)
    d_r = jax.lax.rsqrt(drb_ref[...] + jnp.float32(1e-7))
    hid = jnp.maximum(acc * d_r, jnp.float32(0.0))
    out_ref[...] = jnp.dot(hid, wd_ref[...].T,
                           preferred_element_type=jnp.float32)


def _spmm_head(at_a, at_r, at_b, emb, deg_a, deg_r, deg_b, w_dense,
               block=256):
    n, d_gcn = emb.shape
    h = n // 2
    d_out = w_dense.shape[0]
    common = [
        pl.BlockSpec((n, d_gcn), lambda i: (0, 0)),
        pl.BlockSpec((h, 1), lambda i: (0, 0)),
        pl.BlockSpec((h, 1), lambda i: (0, 0)),
        pl.BlockSpec((h, 1), lambda i: (0, 0)),
    ]
    top = pl.pallas_call(
        _spmm_top_body,
        grid=(h // block,),
        in_specs=[
            pl.BlockSpec((block, h), lambda i: (i, 0)),
            pl.BlockSpec((block, h), lambda i: (i, 0)),
            *common,
            pl.BlockSpec((block, 1), lambda i: (i, 0)),
            pl.BlockSpec((block, 1), lambda i: (i, 0)),
            pl.BlockSpec((d_out, d_gcn), lambda i: (0, 0)),
        ],
        out_specs=pl.BlockSpec((block, d_out), lambda i: (i, 0)),
        out_shape=jax.ShapeDtypeStruct((h, d_out), jnp.float32),
        scratch_shapes=[pltpu.VMEM((n, d_gcn), jnp.float32)],
    )(at_a, at_r, emb, deg_a, deg_r, deg_b, deg_a, deg_r, w_dense)
    bot = pl.pallas_call(
        _spmm_bot_body,
        grid=(h // block,),
        in_specs=[
            pl.BlockSpec((block, n), lambda i: (i, 0)),
            *common,
            pl.BlockSpec((block, 1), lambda i: (i, 0)),
            pl.BlockSpec((d_out, d_gcn), lambda i: (0, 0)),
        ],
        out_specs=pl.BlockSpec((block, d_out), lambda i: (i, 0)),
        out_shape=jax.ShapeDtypeStruct((h, d_out), jnp.float32),
        scratch_shapes=[pltpu.VMEM((n, d_gcn), jnp.float32)],
    )(at_b, emb, deg_a, deg_r, deg_b, deg_b, w_dense)
    return top, bot


def kernel(user_X, item_X, graph_A, mask_sub_adj, P_symm, W_gcn, W_dense):
    num_user = user_X.shape[0]
    x = jnp.concatenate([user_X, item_X], axis=0)
    n = x.shape[0]
    w = n + 8
    block = next(b for b in (1024, 768, 512, 256, 128) if (n // 2) % b == 0)
    nb = n // block
    hb = nb // 2
    h = n // 2

    emb = _embed(x, W_gcn)
    mask8 = mask_sub_adj.astype(jnp.int8)
    lpad_a_flat = _expand_rows_sc(P_symm, n, w, 0, h, block)
    lpad_b_flat = _expand_rows_sc(P_symm, n, w, h, h, block,
                                  dep=jax.lax.slice(lpad_a_flat, (0,), (8,)))
    lpad_a = lpad_a_flat.reshape(h, w)
    lpad_b = lpad_b_flat.reshape(h, w)

    at_a, deg_a = _atilde_part(graph_A, mask8, lpad_a,
                               0, 0, hb, hb, block)
    at_b, deg_b = _atilde_part(graph_A, mask8, lpad_b,
                               hb, 0, hb, nb, block)
    at_r, deg_r = _atilde_part(graph_A, mask8, lpad_b,
                               0, hb, hb, hb, block)

    top, bot = _spmm_head(at_a, at_r, at_b, emb, deg_a, deg_r, deg_b,
                          W_dense)
    out = jnp.concatenate([top, bot], axis=0)
    return out[:num_user], out[num_user:]
